# baseline (speedup 1.0000x reference)
import functools

import jax
import jax.numpy as jnp
from jax import lax
from jax.experimental import pallas as pl
from jax.experimental.pallas import tpu as pltpu
from jax.experimental.pallas import tpu_sc as plsc

N = 10000
E = 160000
NS = 16
NCORES = 2
CHUNK = 128
NCH = 80
EPT = NCH * CHUNK
PADE = NS * EPT - E
NPAD = 10008
RPT = 624
ZL = NPAD - (NS - 1) * RPT
CL = N - (NS - 1) * RPT
RB = 1000

_f32 = jnp.float32
_bf16 = jnp.bfloat16


@functools.cache
def _make_sc_scatter(W):
  mesh = plsc.VectorSubcoreMesh(core_axis_name="c", subcore_axis_name="s",
                                num_cores=NCORES, num_subcores=NS)
  scratch = [
      pltpu.VMEM((NCH, CHUNK), jnp.int32),
      pltpu.VMEM((NCH, CHUNK), jnp.int32),
      pltpu.VMEM((CHUNK, W), _bf16),
      pltpu.VMEM((CHUNK, W), _bf16),
      pltpu.VMEM_SHARED((NPAD, W), _bf16),
      pltpu.SemaphoreType.DMA,
      pltpu.SemaphoreType.DMA,
      pltpu.SemaphoreType.DMA,
      pltpu.SemaphoreType.DMA,
  ]

  def body(t_hbm, srcp_hbm, dstp_hbm, z_hbm, out_hbm,
           src_v, dst_v, buf0, buf1, acc, sem0, sem1, sem2, sem3):
    core = lax.axis_index("c")
    sub = lax.axis_index("s")

    pltpu.sync_copy(srcp_hbm.at[sub], src_v)
    pltpu.sync_copy(dstp_hbm.at[sub], dst_v)

    @pl.when(sub < NS - 1)
    def _():
      pltpu.sync_copy(z_hbm.at[pl.ds(0, RPT)], acc.at[pl.ds(sub * RPT, RPT)])
    @pl.when(sub == NS - 1)
    def _():
      pltpu.sync_copy(z_hbm, acc.at[pl.ds((NS - 1) * RPT, ZL)])
    plsc.subcore_barrier()

    t_half = t_hbm.at[core]

    def chunk(j2, carry):
      e0 = 2 * j2

      @pl.when(j2 > 0)
      def _():
        pltpu.make_async_copy(buf0, acc.at[dst_v.at[0]], sem2).wait()
        pltpu.make_async_copy(buf1, acc.at[dst_v.at[0]], sem3).wait()

      d0 = pltpu.async_copy(t_half.at[src_v.at[e0]], buf0, sem0)
      d1 = pltpu.async_copy(t_half.at[src_v.at[e0 + 1]], buf1, sem1)
      d0.wait()
      pltpu.async_copy(buf0, acc.at[dst_v.at[e0]], sem2, add=True)
      d1.wait()
      pltpu.async_copy(buf1, acc.at[dst_v.at[e0 + 1]], sem3, add=True)
      return carry

    lax.fori_loop(0, NCH // 2, chunk, 0)
    pltpu.make_async_copy(buf0, acc.at[dst_v.at[0]], sem2).wait()
    pltpu.make_async_copy(buf1, acc.at[dst_v.at[0]], sem3).wait()
    plsc.subcore_barrier()

    out_half = out_hbm.at[core]

    @pl.when(sub < NS - 1)
    def _():
      pltpu.sync_copy(acc.at[pl.ds(sub * RPT, RPT)],
                      out_half.at[pl.ds(sub * RPT, RPT)])
    @pl.when(sub == NS - 1)
    def _():
      pltpu.sync_copy(acc.at[pl.ds((NS - 1) * RPT, CL)],
                      out_half.at[pl.ds((NS - 1) * RPT, CL)])

  return pl.kernel(body,
                   out_type=jax.ShapeDtypeStruct((NCORES, N, W), _bf16),
                   mesh=mesh, scratch_types=tuple(scratch),
                   compiler_params=pltpu.CompilerParams(
                       use_tc_tiling_on_sc=False))


@functools.cache
def _make_sc_deg():
  mesh = plsc.VectorSubcoreMesh(core_axis_name="c", subcore_axis_name="s",
                                num_cores=NCORES, num_subcores=NS)
  scratch = [
      pltpu.VMEM((NCH, CHUNK), jnp.int32),
      pltpu.VMEM((CHUNK, 32), _f32),
      pltpu.VMEM_SHARED((NPAD, 32), _f32),
  ]

  def body(dstp_hbm, o_hbm, z8_hbm, deg_hbm, dst_v, ones_v, dacc):
    core = lax.axis_index("c")
    sub = lax.axis_index("s")

    @pl.when(core == 0)
    def _():
      pltpu.sync_copy(dstp_hbm.at[sub], dst_v)
      pltpu.sync_copy(o_hbm, ones_v)

      @pl.when(sub < NS - 1)
      def _():
        pltpu.sync_copy(z8_hbm.at[pl.ds(0, RPT)],
                        dacc.at[pl.ds(sub * RPT, RPT)])
      @pl.when(sub == NS - 1)
      def _():
        pltpu.sync_copy(z8_hbm, dacc.at[pl.ds((NS - 1) * RPT, ZL)])
      plsc.subcore_barrier()

      def chunk(j, carry):
        pltpu.sync_copy(ones_v, dacc.at[dst_v.at[j]], add=True)
        return carry

      lax.fori_loop(0, NCH, chunk, 0)
      plsc.subcore_barrier()

      @pl.when(sub < NS - 1)
      def _():
        pltpu.sync_copy(dacc.at[pl.ds(sub * RPT, RPT)],
                        deg_hbm.at[pl.ds(sub * RPT, RPT)])
      @pl.when(sub == NS - 1)
      def _():
        pltpu.sync_copy(dacc.at[pl.ds((NS - 1) * RPT, CL)],
                        deg_hbm.at[pl.ds((NS - 1) * RPT, CL)])

  return pl.kernel(body,
                   out_type=jax.ShapeDtypeStruct((N, 32), _f32),
                   mesh=mesh, scratch_types=tuple(scratch),
                   compiler_params=pltpu.CompilerParams(
                       use_tc_tiling_on_sc=False))


def _first_body(x_ref, w_ref, b_ref, s_ref, t_ref):
  st = jnp.dot(x_ref[...], w_ref[...], preferred_element_type=_f32)
  h_out = s_ref.shape[1]
  wh = t_ref.shape[2]
  s_ref[...] = st[:, :h_out] + b_ref[...]
  t_ref[...] = jnp.stack(
      [st[:, h_out:h_out + wh], st[:, h_out + wh:]], axis=0).astype(_bf16)


def _mid_body(s_in_ref, a_ref, deg_ref, w_ref, b_ref, s_ref, t_ref):
  inv = 1.0 / jnp.clip(deg_ref[:, 0:1], 1.0, None)
  agg = jnp.concatenate([a_ref[0], a_ref[1]], axis=1).astype(_f32)
  h = jnp.maximum(s_in_ref[...] + agg * inv, 0.0)
  st = jnp.dot(h, w_ref[...], preferred_element_type=_f32)
  h_out = s_ref.shape[1]
  wh = t_ref.shape[2]
  s_ref[...] = st[:, :h_out] + b_ref[...]
  t_ref[...] = jnp.stack(
      [st[:, h_out:h_out + wh], st[:, h_out + wh:]], axis=0).astype(_bf16)


def _out_body(s_in_ref, a_ref, deg_ref, o_ref):
  inv = 1.0 / jnp.clip(deg_ref[:, 0:1], 1.0, None)
  agg = jnp.concatenate([a_ref[0], a_ref[1]], axis=1).astype(_f32)
  o_ref[...] = s_in_ref[...] + agg * inv


def _tc_first(x, wcat, b2d, h_out):
  k = x.shape[1]
  wh = (wcat.shape[1] - h_out) // 2
  return pl.pallas_call(
      _first_body,
      grid=(N // RB,),
      in_specs=[
          pl.BlockSpec((RB, k), lambda i: (i, 0)),
          pl.BlockSpec(wcat.shape, lambda i: (0, 0)),
          pl.BlockSpec((1, h_out), lambda i: (0, 0)),
      ],
      out_specs=[
          pl.BlockSpec((RB, h_out), lambda i: (i, 0)),
          pl.BlockSpec((NCORES, RB, wh), lambda i: (0, i, 0)),
      ],
      out_shape=[
          jax.ShapeDtypeStruct((N, h_out), _f32),
          jax.ShapeDtypeStruct((NCORES, N, wh), _bf16),
      ],
  )(x, wcat, b2d)


def _tc_mid(s_in, agg, deg8, wcat, b2d, h_out):
  k = s_in.shape[1]
  wa = agg.shape[2]
  wh = (wcat.shape[1] - h_out) // 2
  return pl.pallas_call(
      _mid_body,
      grid=(N // RB,),
      in_specs=[
          pl.BlockSpec((RB, k), lambda i: (i, 0)),
          pl.BlockSpec((NCORES, RB, wa), lambda i: (0, i, 0)),
          pl.BlockSpec((RB, 32), lambda i: (i, 0)),
          pl.BlockSpec(wcat.shape, lambda i: (0, 0)),
          pl.BlockSpec((1, h_out), lambda i: (0, 0)),
      ],
      out_specs=[
          pl.BlockSpec((RB, h_out), lambda i: (i, 0)),
          pl.BlockSpec((NCORES, RB, wh), lambda i: (0, i, 0)),
      ],
      out_shape=[
          jax.ShapeDtypeStruct((N, h_out), _f32),
          jax.ShapeDtypeStruct((NCORES, N, wh), _bf16),
      ],
  )(s_in, agg, deg8, wcat, b2d)


def _tc_out(s_in, agg, deg8):
  k = s_in.shape[1]
  wa = agg.shape[2]
  return pl.pallas_call(
      _out_body,
      grid=(N // RB,),
      in_specs=[
          pl.BlockSpec((RB, k), lambda i: (i, 0)),
          pl.BlockSpec((NCORES, RB, wa), lambda i: (0, i, 0)),
          pl.BlockSpec((RB, 32), lambda i: (i, 0)),
      ],
      out_specs=pl.BlockSpec((RB, k), lambda i: (i, 0)),
      out_shape=jax.ShapeDtypeStruct((N, k), _f32),
  )(s_in, agg, deg8)


@jax.jit
def kernel(x, edge_index, W_self1, W_neigh1, b1, W_self2, W_neigh2, b2,
           W_self3, W_neigh3, b3):
  src = edge_index[0].astype(jnp.int32)
  dst = edge_index[1].astype(jnp.int32)
  srcp = jnp.concatenate([src, jnp.zeros((PADE,), jnp.int32)]).reshape(NS, NCH, CHUNK)
  dstp = jnp.concatenate([dst, jnp.full((PADE,), N, jnp.int32)]).reshape(NS, NCH, CHUNK)
  z128 = jnp.zeros((ZL, 128), _bf16)
  z32 = jnp.zeros((ZL, 32), _bf16)
  z8 = jnp.zeros((ZL, 32), _f32)
  o8 = jnp.ones((CHUNK, 32), _f32)

  wc1 = jnp.concatenate([W_self1, W_neigh1], axis=1)
  wc2 = jnp.concatenate([W_self2, W_neigh2], axis=1)
  wc3 = jnp.concatenate([W_self3, W_neigh3], axis=1)

  deg8 = _make_sc_deg()(dstp, o8, z8)
  s1, t1 = _tc_first(x, wc1, b1[None], 256)
  agg1 = _make_sc_scatter(128)(t1, srcp, dstp, z128)
  s2, t2 = _tc_mid(s1, agg1, deg8, wc2, b2[None], 256)
  agg2 = _make_sc_scatter(128)(t2, srcp, dstp, z128)
  s3, t3 = _tc_mid(s2, agg2, deg8, wc3, b3[None], 64)
  agg3 = _make_sc_scatter(32)(t3, srcp, dstp, z32)
  return _tc_out(s3, agg3, deg8)


# speedup vs baseline: 1.0600x; 1.0600x over previous
import functools

import jax
import jax.numpy as jnp
from jax import lax
from jax.experimental import pallas as pl
from jax.experimental.pallas import tpu as pltpu
from jax.experimental.pallas import tpu_sc as plsc

N = 10000
E = 160000
NS = 16
NCORES = 2
CHUNK = 96
NCH = 106
EPT = NCH * CHUNK
PADE = NS * EPT - E
NPAD = 10008
RPT = 624
ZL = NPAD - (NS - 1) * RPT
CL = N - (NS - 1) * RPT
RB = 1000

_f32 = jnp.float32
_bf16 = jnp.bfloat16


@functools.cache
def _make_sc_scatter(W):
  mesh = plsc.VectorSubcoreMesh(core_axis_name="c", subcore_axis_name="s",
                                num_cores=NCORES, num_subcores=NS)
  scratch = [
      pltpu.VMEM((NCH, CHUNK), jnp.int32),
      pltpu.VMEM((NCH, CHUNK), jnp.int32),
      pltpu.VMEM((CHUNK, W), _bf16),
      pltpu.VMEM((CHUNK, W), _bf16),
      pltpu.VMEM_SHARED((NPAD, W), _bf16),
      pltpu.SemaphoreType.DMA,
      pltpu.SemaphoreType.DMA,
      pltpu.SemaphoreType.DMA,
      pltpu.SemaphoreType.DMA,
  ]

  def body(t_hbm, srcp_hbm, dstp_hbm, z_hbm, out_hbm,
           src_v, dst_v, buf0, buf1, acc, sem0, sem1, sem2, sem3):
    core = lax.axis_index("c")
    sub = lax.axis_index("s")

    pltpu.sync_copy(srcp_hbm.at[sub], src_v)
    pltpu.sync_copy(dstp_hbm.at[sub], dst_v)

    @pl.when(sub < NS - 1)
    def _():
      pltpu.sync_copy(z_hbm.at[pl.ds(0, RPT)], acc.at[pl.ds(sub * RPT, RPT)])
    @pl.when(sub == NS - 1)
    def _():
      pltpu.sync_copy(z_hbm, acc.at[pl.ds((NS - 1) * RPT, ZL)])
    plsc.subcore_barrier()

    t_half = t_hbm.at[core]

    def chunk(j2, carry):
      e0 = 2 * j2

      @pl.when(j2 > 0)
      def _():
        pltpu.make_async_copy(buf0, acc.at[dst_v.at[0]], sem2).wait()
        pltpu.make_async_copy(buf1, acc.at[dst_v.at[0]], sem3).wait()

      d0 = pltpu.async_copy(t_half.at[src_v.at[e0]], buf0, sem0)
      d1 = pltpu.async_copy(t_half.at[src_v.at[e0 + 1]], buf1, sem1)
      d0.wait()
      pltpu.async_copy(buf0, acc.at[dst_v.at[e0]], sem2, add=True)
      d1.wait()
      pltpu.async_copy(buf1, acc.at[dst_v.at[e0 + 1]], sem3, add=True)
      return carry

    lax.fori_loop(0, NCH // 2, chunk, 0)
    pltpu.make_async_copy(buf0, acc.at[dst_v.at[0]], sem2).wait()
    pltpu.make_async_copy(buf1, acc.at[dst_v.at[0]], sem3).wait()
    plsc.subcore_barrier()

    out_half = out_hbm.at[core]

    @pl.when(sub < NS - 1)
    def _():
      pltpu.sync_copy(acc.at[pl.ds(sub * RPT, RPT)],
                      out_half.at[pl.ds(sub * RPT, RPT)])
    @pl.when(sub == NS - 1)
    def _():
      pltpu.sync_copy(acc.at[pl.ds((NS - 1) * RPT, CL)],
                      out_half.at[pl.ds((NS - 1) * RPT, CL)])

  return pl.kernel(body,
                   out_type=jax.ShapeDtypeStruct((NCORES, N, W), _bf16),
                   mesh=mesh, scratch_types=tuple(scratch),
                   compiler_params=pltpu.CompilerParams(
                       use_tc_tiling_on_sc=False))


@functools.cache
def _make_sc_deg():
  mesh = plsc.VectorSubcoreMesh(core_axis_name="c", subcore_axis_name="s",
                                num_cores=NCORES, num_subcores=NS)
  scratch = [
      pltpu.VMEM((NCH, CHUNK), jnp.int32),
      pltpu.VMEM((CHUNK, 32), _f32),
      pltpu.VMEM_SHARED((NPAD, 32), _f32),
  ]

  def body(dstp_hbm, o_hbm, z8_hbm, deg_hbm, dst_v, ones_v, dacc):
    core = lax.axis_index("c")
    sub = lax.axis_index("s")

    @pl.when(core == 0)
    def _():
      pltpu.sync_copy(dstp_hbm.at[sub], dst_v)
      pltpu.sync_copy(o_hbm, ones_v)

      @pl.when(sub < NS - 1)
      def _():
        pltpu.sync_copy(z8_hbm.at[pl.ds(0, RPT)],
                        dacc.at[pl.ds(sub * RPT, RPT)])
      @pl.when(sub == NS - 1)
      def _():
        pltpu.sync_copy(z8_hbm, dacc.at[pl.ds((NS - 1) * RPT, ZL)])
      plsc.subcore_barrier()

      def chunk(j, carry):
        pltpu.sync_copy(ones_v, dacc.at[dst_v.at[j]], add=True)
        return carry

      lax.fori_loop(0, NCH, chunk, 0)
      plsc.subcore_barrier()

      @pl.when(sub < NS - 1)
      def _():
        pltpu.sync_copy(dacc.at[pl.ds(sub * RPT, RPT)],
                        deg_hbm.at[pl.ds(sub * RPT, RPT)])
      @pl.when(sub == NS - 1)
      def _():
        pltpu.sync_copy(dacc.at[pl.ds((NS - 1) * RPT, CL)],
                        deg_hbm.at[pl.ds((NS - 1) * RPT, CL)])

  return pl.kernel(body,
                   out_type=jax.ShapeDtypeStruct((N, 32), _f32),
                   mesh=mesh, scratch_types=tuple(scratch),
                   compiler_params=pltpu.CompilerParams(
                       use_tc_tiling_on_sc=False))


def _first_body(x_ref, w_ref, b_ref, s_ref, t_ref):
  st = jnp.dot(x_ref[...], w_ref[...], preferred_element_type=_f32)
  h_out = s_ref.shape[1]
  wh = t_ref.shape[2]
  s_ref[...] = st[:, :h_out] + b_ref[...]
  t_ref[...] = jnp.stack(
      [st[:, h_out:h_out + wh], st[:, h_out + wh:]], axis=0).astype(_bf16)


def _mid_body(s_in_ref, a_ref, deg_ref, w_ref, b_ref, s_ref, t_ref):
  inv = 1.0 / jnp.clip(deg_ref[:, 0:1], 1.0, None)
  agg = jnp.concatenate([a_ref[0], a_ref[1]], axis=1).astype(_f32)
  h = jnp.maximum(s_in_ref[...] + agg * inv, 0.0)
  st = jnp.dot(h, w_ref[...], preferred_element_type=_f32)
  h_out = s_ref.shape[1]
  wh = t_ref.shape[2]
  s_ref[...] = st[:, :h_out] + b_ref[...]
  t_ref[...] = jnp.stack(
      [st[:, h_out:h_out + wh], st[:, h_out + wh:]], axis=0).astype(_bf16)


def _out_body(s_in_ref, a_ref, deg_ref, o_ref):
  inv = 1.0 / jnp.clip(deg_ref[:, 0:1], 1.0, None)
  agg = jnp.concatenate([a_ref[0], a_ref[1]], axis=1).astype(_f32)
  o_ref[...] = s_in_ref[...] + agg * inv


def _tc_first(x, wcat, b2d, h_out):
  k = x.shape[1]
  wh = (wcat.shape[1] - h_out) // 2
  return pl.pallas_call(
      _first_body,
      grid=(N // RB,),
      in_specs=[
          pl.BlockSpec((RB, k), lambda i: (i, 0)),
          pl.BlockSpec(wcat.shape, lambda i: (0, 0)),
          pl.BlockSpec((1, h_out), lambda i: (0, 0)),
      ],
      out_specs=[
          pl.BlockSpec((RB, h_out), lambda i: (i, 0)),
          pl.BlockSpec((NCORES, RB, wh), lambda i: (0, i, 0)),
      ],
      out_shape=[
          jax.ShapeDtypeStruct((N, h_out), _f32),
          jax.ShapeDtypeStruct((NCORES, N, wh), _bf16),
      ],
  )(x, wcat, b2d)


def _tc_mid(s_in, agg, deg8, wcat, b2d, h_out):
  k = s_in.shape[1]
  wa = agg.shape[2]
  wh = (wcat.shape[1] - h_out) // 2
  return pl.pallas_call(
      _mid_body,
      grid=(N // RB,),
      in_specs=[
          pl.BlockSpec((RB, k), lambda i: (i, 0)),
          pl.BlockSpec((NCORES, RB, wa), lambda i: (0, i, 0)),
          pl.BlockSpec((RB, 32), lambda i: (i, 0)),
          pl.BlockSpec(wcat.shape, lambda i: (0, 0)),
          pl.BlockSpec((1, h_out), lambda i: (0, 0)),
      ],
      out_specs=[
          pl.BlockSpec((RB, h_out), lambda i: (i, 0)),
          pl.BlockSpec((NCORES, RB, wh), lambda i: (0, i, 0)),
      ],
      out_shape=[
          jax.ShapeDtypeStruct((N, h_out), _f32),
          jax.ShapeDtypeStruct((NCORES, N, wh), _bf16),
      ],
  )(s_in, agg, deg8, wcat, b2d)


def _tc_out(s_in, agg, deg8):
  k = s_in.shape[1]
  wa = agg.shape[2]
  return pl.pallas_call(
      _out_body,
      grid=(N // RB,),
      in_specs=[
          pl.BlockSpec((RB, k), lambda i: (i, 0)),
          pl.BlockSpec((NCORES, RB, wa), lambda i: (0, i, 0)),
          pl.BlockSpec((RB, 32), lambda i: (i, 0)),
      ],
      out_specs=pl.BlockSpec((RB, k), lambda i: (i, 0)),
      out_shape=jax.ShapeDtypeStruct((N, k), _f32),
  )(s_in, agg, deg8)


@jax.jit
def kernel(x, edge_index, W_self1, W_neigh1, b1, W_self2, W_neigh2, b2,
           W_self3, W_neigh3, b3):
  src = edge_index[0].astype(jnp.int32)
  dst = edge_index[1].astype(jnp.int32)
  srcp = jnp.concatenate([src, jnp.zeros((PADE,), jnp.int32)]).reshape(NS, NCH, CHUNK)
  dstp = jnp.concatenate([dst, jnp.full((PADE,), N, jnp.int32)]).reshape(NS, NCH, CHUNK)
  z128 = jnp.zeros((ZL, 128), _bf16)
  z32 = jnp.zeros((ZL, 32), _bf16)
  z8 = jnp.zeros((ZL, 32), _f32)
  o8 = jnp.ones((CHUNK, 32), _f32)

  wc1 = jnp.concatenate([W_self1, W_neigh1], axis=1)
  wc2 = jnp.concatenate([W_self2, W_neigh2], axis=1)
  wc3 = jnp.concatenate([W_self3, W_neigh3], axis=1)

  deg8 = _make_sc_deg()(dstp, o8, z8)
  s1, t1 = _tc_first(x, wc1, b1[None], 256)
  agg1 = _make_sc_scatter(128)(t1, srcp, dstp, z128)
  s2, t2 = _tc_mid(s1, agg1, deg8, wc2, b2[None], 256)
  agg2 = _make_sc_scatter(128)(t2, srcp, dstp, z128)
  s3, t3 = _tc_mid(s2, agg2, deg8, wc3, b3[None], 64)
  agg3 = _make_sc_scatter(32)(t3, srcp, dstp, z32)
  return _tc_out(s3, agg3, deg8)


# speedup vs baseline: 1.2724x; 1.2003x over previous
import functools

import jax
import jax.numpy as jnp
from jax import lax
from jax.experimental import pallas as pl
from jax.experimental.pallas import tpu as pltpu
from jax.experimental.pallas import tpu_sc as plsc

N = 10000
E = 160000
NS = 16
NCORES = 2
CHUNK = 112
NCH = 90
EPT = NCH * CHUNK
PADE = NS * EPT - E
NPAD = 10008
RPT = 624
ZL = NPAD - (NS - 1) * RPT
CL = N - (NS - 1) * RPT
RB = 1000

_f32 = jnp.float32
_bf16 = jnp.bfloat16


@functools.cache
def _make_sc_scatter(W, with_deg=False):
  mesh = plsc.VectorSubcoreMesh(core_axis_name="c", subcore_axis_name="s",
                                num_cores=NCORES, num_subcores=NS)
  scratch = [
      pltpu.VMEM((NCH, CHUNK), jnp.int32),
      pltpu.VMEM((NCH, CHUNK), jnp.int32),
      pltpu.VMEM((CHUNK, W), _bf16),
      pltpu.VMEM((CHUNK, W), _bf16),
      pltpu.VMEM_SHARED((NPAD, W), _bf16),
      pltpu.SemaphoreType.DMA,
      pltpu.SemaphoreType.DMA,
      pltpu.SemaphoreType.DMA,
      pltpu.SemaphoreType.DMA,
  ]
  outs = [jax.ShapeDtypeStruct((NCORES, N, W), _bf16)]
  if with_deg:
    outs.append(jax.ShapeDtypeStruct((N, 32), _f32))
    scratch += [
        pltpu.VMEM((CHUNK, 32), _f32),
        pltpu.VMEM_SHARED((NPAD, 32), _f32),
        pltpu.SemaphoreType.DMA,
    ]

  def body(t_hbm, srcp_hbm, dstp_hbm, z_hbm, *rest):
    if with_deg:
      (o_hbm, zd_hbm, out_hbm, deg_hbm, src_v, dst_v, buf0, buf1, acc,
       sem0, sem1, sem2, sem3, ones_v, dacc, sem4) = rest
    else:
      (out_hbm, src_v, dst_v, buf0, buf1, acc,
       sem0, sem1, sem2, sem3) = rest
    core = lax.axis_index("c")
    sub = lax.axis_index("s")

    pltpu.sync_copy(srcp_hbm.at[sub], src_v)
    pltpu.sync_copy(dstp_hbm.at[sub], dst_v)

    @pl.when(sub < NS - 1)
    def _():
      pltpu.sync_copy(z_hbm.at[pl.ds(0, RPT)], acc.at[pl.ds(sub * RPT, RPT)])
    @pl.when(sub == NS - 1)
    def _():
      pltpu.sync_copy(z_hbm, acc.at[pl.ds((NS - 1) * RPT, ZL)])
    if with_deg:
      @pl.when(core == 0)
      def _():
        pltpu.sync_copy(o_hbm, ones_v)

        @pl.when(sub < NS - 1)
        def _():
          pltpu.sync_copy(zd_hbm.at[pl.ds(0, RPT)],
                          dacc.at[pl.ds(sub * RPT, RPT)])
        @pl.when(sub == NS - 1)
        def _():
          pltpu.sync_copy(zd_hbm, dacc.at[pl.ds((NS - 1) * RPT, ZL)])
    plsc.subcore_barrier()

    t_half = t_hbm.at[core]

    def chunk(j2, carry):
      e0 = 2 * j2

      @pl.when(j2 > 0)
      def _():
        pltpu.make_async_copy(buf0, acc.at[dst_v.at[0]], sem2).wait()
        pltpu.make_async_copy(buf1, acc.at[dst_v.at[0]], sem3).wait()
      if with_deg:
        @pl.when((core == 0) & (j2 > 0))
        def _():
          pltpu.make_async_copy(ones_v, dacc.at[dst_v.at[0]], sem4).wait()
          pltpu.make_async_copy(ones_v, dacc.at[dst_v.at[0]], sem4).wait()

      d0 = pltpu.async_copy(t_half.at[src_v.at[e0]], buf0, sem0)
      d1 = pltpu.async_copy(t_half.at[src_v.at[e0 + 1]], buf1, sem1)
      if with_deg:
        @pl.when(core == 0)
        def _():
          pltpu.async_copy(ones_v, dacc.at[dst_v.at[e0]], sem4, add=True)
          pltpu.async_copy(ones_v, dacc.at[dst_v.at[e0 + 1]], sem4, add=True)
      d0.wait()
      pltpu.async_copy(buf0, acc.at[dst_v.at[e0]], sem2, add=True)
      d1.wait()
      pltpu.async_copy(buf1, acc.at[dst_v.at[e0 + 1]], sem3, add=True)
      return carry

    lax.fori_loop(0, NCH // 2, chunk, 0)
    pltpu.make_async_copy(buf0, acc.at[dst_v.at[0]], sem2).wait()
    pltpu.make_async_copy(buf1, acc.at[dst_v.at[0]], sem3).wait()
    if with_deg:
      @pl.when(core == 0)
      def _():
        pltpu.make_async_copy(ones_v, dacc.at[dst_v.at[0]], sem4).wait()
        pltpu.make_async_copy(ones_v, dacc.at[dst_v.at[0]], sem4).wait()
    plsc.subcore_barrier()

    out_half = out_hbm.at[core]

    @pl.when(sub < NS - 1)
    def _():
      pltpu.sync_copy(acc.at[pl.ds(sub * RPT, RPT)],
                      out_half.at[pl.ds(sub * RPT, RPT)])
    @pl.when(sub == NS - 1)
    def _():
      pltpu.sync_copy(acc.at[pl.ds((NS - 1) * RPT, CL)],
                      out_half.at[pl.ds((NS - 1) * RPT, CL)])
    if with_deg:
      @pl.when((core == 0) & (sub < NS - 1))
      def _():
        pltpu.sync_copy(dacc.at[pl.ds(sub * RPT, RPT)],
                        deg_hbm.at[pl.ds(sub * RPT, RPT)])
      @pl.when((core == 0) & (sub == NS - 1))
      def _():
        pltpu.sync_copy(dacc.at[pl.ds((NS - 1) * RPT, CL)],
                        deg_hbm.at[pl.ds((NS - 1) * RPT, CL)])

  return pl.kernel(body, out_type=tuple(outs),
                   mesh=mesh, scratch_types=tuple(scratch),
                   compiler_params=pltpu.CompilerParams(
                       use_tc_tiling_on_sc=False))


@functools.cache
def _make_sc_deg():
  mesh = plsc.VectorSubcoreMesh(core_axis_name="c", subcore_axis_name="s",
                                num_cores=NCORES, num_subcores=NS)
  scratch = [
      pltpu.VMEM((NCH, CHUNK), jnp.int32),
      pltpu.VMEM((CHUNK, 32), _f32),
      pltpu.VMEM_SHARED((NPAD, 32), _f32),
  ]

  def body(dstp_hbm, o_hbm, z8_hbm, deg_hbm, dst_v, ones_v, dacc):
    core = lax.axis_index("c")
    sub = lax.axis_index("s")

    @pl.when(core == 0)
    def _():
      pltpu.sync_copy(dstp_hbm.at[sub], dst_v)
      pltpu.sync_copy(o_hbm, ones_v)

      @pl.when(sub < NS - 1)
      def _():
        pltpu.sync_copy(z8_hbm.at[pl.ds(0, RPT)],
                        dacc.at[pl.ds(sub * RPT, RPT)])
      @pl.when(sub == NS - 1)
      def _():
        pltpu.sync_copy(z8_hbm, dacc.at[pl.ds((NS - 1) * RPT, ZL)])
      plsc.subcore_barrier()

      def chunk(j, carry):
        pltpu.sync_copy(ones_v, dacc.at[dst_v.at[j]], add=True)
        return carry

      lax.fori_loop(0, NCH, chunk, 0)
      plsc.subcore_barrier()

      @pl.when(sub < NS - 1)
      def _():
        pltpu.sync_copy(dacc.at[pl.ds(sub * RPT, RPT)],
                        deg_hbm.at[pl.ds(sub * RPT, RPT)])
      @pl.when(sub == NS - 1)
      def _():
        pltpu.sync_copy(dacc.at[pl.ds((NS - 1) * RPT, CL)],
                        deg_hbm.at[pl.ds((NS - 1) * RPT, CL)])

  return pl.kernel(body,
                   out_type=jax.ShapeDtypeStruct((N, 32), _f32),
                   mesh=mesh, scratch_types=tuple(scratch),
                   compiler_params=pltpu.CompilerParams(
                       use_tc_tiling_on_sc=False))


def _first_body(x_ref, w_ref, b_ref, s_ref, t_ref):
  st = jnp.dot(x_ref[...], w_ref[...], preferred_element_type=_f32)
  h_out = s_ref.shape[1]
  wh = t_ref.shape[2]
  s_ref[...] = st[:, :h_out] + b_ref[...]
  t_ref[...] = jnp.stack(
      [st[:, h_out:h_out + wh], st[:, h_out + wh:]], axis=0).astype(_bf16)


def _mid_body(s_in_ref, a_ref, deg_ref, w_ref, b_ref, s_ref, t_ref):
  inv = 1.0 / jnp.clip(deg_ref[:, 0:1], 1.0, None)
  agg = jnp.concatenate([a_ref[0], a_ref[1]], axis=1).astype(_f32)
  h = jnp.maximum(s_in_ref[...] + agg * inv, 0.0)
  st = jnp.dot(h, w_ref[...], preferred_element_type=_f32)
  h_out = s_ref.shape[1]
  wh = t_ref.shape[2]
  s_ref[...] = st[:, :h_out] + b_ref[...]
  t_ref[...] = jnp.stack(
      [st[:, h_out:h_out + wh], st[:, h_out + wh:]], axis=0).astype(_bf16)


def _out_body(s_in_ref, a_ref, deg_ref, o_ref):
  inv = 1.0 / jnp.clip(deg_ref[:, 0:1], 1.0, None)
  agg = jnp.concatenate([a_ref[0], a_ref[1]], axis=1).astype(_f32)
  o_ref[...] = s_in_ref[...] + agg * inv


def _tc_first(x, wcat, b2d, h_out):
  k = x.shape[1]
  wh = (wcat.shape[1] - h_out) // 2
  return pl.pallas_call(
      _first_body,
      grid=(N // RB,),
      in_specs=[
          pl.BlockSpec((RB, k), lambda i: (i, 0)),
          pl.BlockSpec(wcat.shape, lambda i: (0, 0)),
          pl.BlockSpec((1, h_out), lambda i: (0, 0)),
      ],
      out_specs=[
          pl.BlockSpec((RB, h_out), lambda i: (i, 0)),
          pl.BlockSpec((NCORES, RB, wh), lambda i: (0, i, 0)),
      ],
      out_shape=[
          jax.ShapeDtypeStruct((N, h_out), _f32),
          jax.ShapeDtypeStruct((NCORES, N, wh), _bf16),
      ],
  )(x, wcat, b2d)


def _tc_mid(s_in, agg, deg8, wcat, b2d, h_out):
  k = s_in.shape[1]
  wa = agg.shape[2]
  wh = (wcat.shape[1] - h_out) // 2
  return pl.pallas_call(
      _mid_body,
      grid=(N // RB,),
      in_specs=[
          pl.BlockSpec((RB, k), lambda i: (i, 0)),
          pl.BlockSpec((NCORES, RB, wa), lambda i: (0, i, 0)),
          pl.BlockSpec((RB, 32), lambda i: (i, 0)),
          pl.BlockSpec(wcat.shape, lambda i: (0, 0)),
          pl.BlockSpec((1, h_out), lambda i: (0, 0)),
      ],
      out_specs=[
          pl.BlockSpec((RB, h_out), lambda i: (i, 0)),
          pl.BlockSpec((NCORES, RB, wh), lambda i: (0, i, 0)),
      ],
      out_shape=[
          jax.ShapeDtypeStruct((N, h_out), _f32),
          jax.ShapeDtypeStruct((NCORES, N, wh), _bf16),
      ],
  )(s_in, agg, deg8, wcat, b2d)


def _tc_out(s_in, agg, deg8):
  k = s_in.shape[1]
  wa = agg.shape[2]
  return pl.pallas_call(
      _out_body,
      grid=(N // RB,),
      in_specs=[
          pl.BlockSpec((RB, k), lambda i: (i, 0)),
          pl.BlockSpec((NCORES, RB, wa), lambda i: (0, i, 0)),
          pl.BlockSpec((RB, 32), lambda i: (i, 0)),
      ],
      out_specs=pl.BlockSpec((RB, k), lambda i: (i, 0)),
      out_shape=jax.ShapeDtypeStruct((N, k), _f32),
  )(s_in, agg, deg8)


@jax.jit
def kernel(x, edge_index, W_self1, W_neigh1, b1, W_self2, W_neigh2, b2,
           W_self3, W_neigh3, b3):
  src = edge_index[0].astype(jnp.int32)
  dst = edge_index[1].astype(jnp.int32)
  srcp = jnp.concatenate([src, jnp.zeros((PADE,), jnp.int32)]).reshape(NS, NCH, CHUNK)
  dstp = jnp.concatenate([dst, jnp.full((PADE,), N, jnp.int32)]).reshape(NS, NCH, CHUNK)
  z128 = jnp.zeros((ZL, 128), _bf16)
  z32 = jnp.zeros((ZL, 32), _bf16)
  z8 = jnp.zeros((ZL, 32), _f32)
  o8 = jnp.ones((CHUNK, 32), _f32)

  wc1 = jnp.concatenate([W_self1, W_neigh1], axis=1)
  wc2 = jnp.concatenate([W_self2, W_neigh2], axis=1)
  wc3 = jnp.concatenate([W_self3, W_neigh3], axis=1)

  s1, t1 = _tc_first(x, wc1, b1[None], 256)
  agg1, deg8 = _make_sc_scatter(128, True)(t1, srcp, dstp, z128, o8, z8)
  s2, t2 = _tc_mid(s1, agg1, deg8, wc2, b2[None], 256)
  (agg2,) = _make_sc_scatter(128)(t2, srcp, dstp, z128)
  s3, t3 = _tc_mid(s2, agg2, deg8, wc3, b3[None], 64)
  (agg3,) = _make_sc_scatter(32)(t3, srcp, dstp, z32)
  return _tc_out(s3, agg3, deg8)


# speedup vs baseline: 1.2865x; 1.0111x over previous
import functools

import jax
import jax.numpy as jnp
from jax import lax
from jax.experimental import pallas as pl
from jax.experimental.pallas import tpu as pltpu
from jax.experimental.pallas import tpu_sc as plsc

N = 10000
E = 160000
NS = 16
NCORES = 2
CHUNK = 120
NCH = 84
EPT = NCH * CHUNK
PADE = NS * EPT - E
NPAD = 10008
RPT = 624
ZL = NPAD - (NS - 1) * RPT
CL = N - (NS - 1) * RPT
RB = 1000

_f32 = jnp.float32
_bf16 = jnp.bfloat16


@functools.cache
def _make_sc_scatter(W, with_deg=False):
  mesh = plsc.VectorSubcoreMesh(core_axis_name="c", subcore_axis_name="s",
                                num_cores=NCORES, num_subcores=NS)
  scratch = [
      pltpu.VMEM((NCH, CHUNK), jnp.int32),
      pltpu.VMEM((NCH, CHUNK), jnp.int32),
      pltpu.VMEM((CHUNK, W), _bf16),
      pltpu.VMEM((CHUNK, W), _bf16),
      pltpu.VMEM_SHARED((NPAD, W), _bf16),
      pltpu.SemaphoreType.DMA,
      pltpu.SemaphoreType.DMA,
      pltpu.SemaphoreType.DMA,
      pltpu.SemaphoreType.DMA,
  ]
  outs = [jax.ShapeDtypeStruct((NCORES, N, W), _bf16)]
  if with_deg:
    outs.append(jax.ShapeDtypeStruct((N, 32), _f32))
    scratch += [
        pltpu.VMEM((CHUNK, 32), _f32),
        pltpu.VMEM_SHARED((NPAD, 32), _f32),
        pltpu.SemaphoreType.DMA,
    ]

  def body(t_hbm, srcp_hbm, dstp_hbm, z_hbm, *rest):
    if with_deg:
      (o_hbm, zd_hbm, out_hbm, deg_hbm, src_v, dst_v, buf0, buf1, acc,
       sem0, sem1, sem2, sem3, ones_v, dacc, sem4) = rest
    else:
      (out_hbm, src_v, dst_v, buf0, buf1, acc,
       sem0, sem1, sem2, sem3) = rest
    core = lax.axis_index("c")
    sub = lax.axis_index("s")

    pltpu.sync_copy(srcp_hbm.at[sub], src_v)
    pltpu.sync_copy(dstp_hbm.at[sub], dst_v)

    @pl.when(sub < NS - 1)
    def _():
      pltpu.sync_copy(z_hbm.at[pl.ds(0, RPT)], acc.at[pl.ds(sub * RPT, RPT)])
    @pl.when(sub == NS - 1)
    def _():
      pltpu.sync_copy(z_hbm, acc.at[pl.ds((NS - 1) * RPT, ZL)])
    if with_deg:
      @pl.when(core == 0)
      def _():
        pltpu.sync_copy(o_hbm, ones_v)

        @pl.when(sub < NS - 1)
        def _():
          pltpu.sync_copy(zd_hbm.at[pl.ds(0, RPT)],
                          dacc.at[pl.ds(sub * RPT, RPT)])
        @pl.when(sub == NS - 1)
        def _():
          pltpu.sync_copy(zd_hbm, dacc.at[pl.ds((NS - 1) * RPT, ZL)])
    plsc.subcore_barrier()

    t_half = t_hbm.at[core]

    def chunk(j2, carry):
      e0 = 2 * j2

      @pl.when(j2 > 0)
      def _():
        pltpu.make_async_copy(buf0, acc.at[dst_v.at[0]], sem2).wait()
        pltpu.make_async_copy(buf1, acc.at[dst_v.at[0]], sem3).wait()
      if with_deg:
        @pl.when((core == 0) & (j2 > 0))
        def _():
          pltpu.make_async_copy(ones_v, dacc.at[dst_v.at[0]], sem4).wait()
          pltpu.make_async_copy(ones_v, dacc.at[dst_v.at[0]], sem4).wait()

      d0 = pltpu.async_copy(t_half.at[src_v.at[e0]], buf0, sem0)
      d1 = pltpu.async_copy(t_half.at[src_v.at[e0 + 1]], buf1, sem1)
      if with_deg:
        @pl.when(core == 0)
        def _():
          pltpu.async_copy(ones_v, dacc.at[dst_v.at[e0]], sem4, add=True)
          pltpu.async_copy(ones_v, dacc.at[dst_v.at[e0 + 1]], sem4, add=True)
      d0.wait()
      pltpu.async_copy(buf0, acc.at[dst_v.at[e0]], sem2, add=True)
      d1.wait()
      pltpu.async_copy(buf1, acc.at[dst_v.at[e0 + 1]], sem3, add=True)
      return carry

    lax.fori_loop(0, NCH // 2, chunk, 0)
    pltpu.make_async_copy(buf0, acc.at[dst_v.at[0]], sem2).wait()
    pltpu.make_async_copy(buf1, acc.at[dst_v.at[0]], sem3).wait()
    if with_deg:
      @pl.when(core == 0)
      def _():
        pltpu.make_async_copy(ones_v, dacc.at[dst_v.at[0]], sem4).wait()
        pltpu.make_async_copy(ones_v, dacc.at[dst_v.at[0]], sem4).wait()
    plsc.subcore_barrier()

    out_half = out_hbm.at[core]

    @pl.when(sub < NS - 1)
    def _():
      pltpu.sync_copy(acc.at[pl.ds(sub * RPT, RPT)],
                      out_half.at[pl.ds(sub * RPT, RPT)])
    @pl.when(sub == NS - 1)
    def _():
      pltpu.sync_copy(acc.at[pl.ds((NS - 1) * RPT, CL)],
                      out_half.at[pl.ds((NS - 1) * RPT, CL)])
    if with_deg:
      @pl.when((core == 0) & (sub < NS - 1))
      def _():
        pltpu.sync_copy(dacc.at[pl.ds(sub * RPT, RPT)],
                        deg_hbm.at[pl.ds(sub * RPT, RPT)])
      @pl.when((core == 0) & (sub == NS - 1))
      def _():
        pltpu.sync_copy(dacc.at[pl.ds((NS - 1) * RPT, CL)],
                        deg_hbm.at[pl.ds((NS - 1) * RPT, CL)])

  return pl.kernel(body, out_type=tuple(outs),
                   mesh=mesh, scratch_types=tuple(scratch),
                   compiler_params=pltpu.CompilerParams(
                       use_tc_tiling_on_sc=False))


@functools.cache
def _make_sc_deg():
  mesh = plsc.VectorSubcoreMesh(core_axis_name="c", subcore_axis_name="s",
                                num_cores=NCORES, num_subcores=NS)
  scratch = [
      pltpu.VMEM((NCH, CHUNK), jnp.int32),
      pltpu.VMEM((CHUNK, 32), _f32),
      pltpu.VMEM_SHARED((NPAD, 32), _f32),
  ]

  def body(dstp_hbm, o_hbm, z8_hbm, deg_hbm, dst_v, ones_v, dacc):
    core = lax.axis_index("c")
    sub = lax.axis_index("s")

    @pl.when(core == 0)
    def _():
      pltpu.sync_copy(dstp_hbm.at[sub], dst_v)
      pltpu.sync_copy(o_hbm, ones_v)

      @pl.when(sub < NS - 1)
      def _():
        pltpu.sync_copy(z8_hbm.at[pl.ds(0, RPT)],
                        dacc.at[pl.ds(sub * RPT, RPT)])
      @pl.when(sub == NS - 1)
      def _():
        pltpu.sync_copy(z8_hbm, dacc.at[pl.ds((NS - 1) * RPT, ZL)])
      plsc.subcore_barrier()

      def chunk(j, carry):
        pltpu.sync_copy(ones_v, dacc.at[dst_v.at[j]], add=True)
        return carry

      lax.fori_loop(0, NCH, chunk, 0)
      plsc.subcore_barrier()

      @pl.when(sub < NS - 1)
      def _():
        pltpu.sync_copy(dacc.at[pl.ds(sub * RPT, RPT)],
                        deg_hbm.at[pl.ds(sub * RPT, RPT)])
      @pl.when(sub == NS - 1)
      def _():
        pltpu.sync_copy(dacc.at[pl.ds((NS - 1) * RPT, CL)],
                        deg_hbm.at[pl.ds((NS - 1) * RPT, CL)])

  return pl.kernel(body,
                   out_type=jax.ShapeDtypeStruct((N, 32), _f32),
                   mesh=mesh, scratch_types=tuple(scratch),
                   compiler_params=pltpu.CompilerParams(
                       use_tc_tiling_on_sc=False))


def _first_body(x_ref, w_ref, b_ref, s_ref, t_ref):
  st = jnp.dot(x_ref[...], w_ref[...], preferred_element_type=_f32)
  h_out = s_ref.shape[1]
  wh = t_ref.shape[2]
  s_ref[...] = st[:, :h_out] + b_ref[...]
  t_ref[...] = jnp.stack(
      [st[:, h_out:h_out + wh], st[:, h_out + wh:]], axis=0).astype(_bf16)


def _mid_body(s_in_ref, a_ref, deg_ref, w_ref, b_ref, s_ref, t_ref):
  inv = 1.0 / jnp.clip(deg_ref[:, 0:1], 1.0, None)
  agg = jnp.concatenate([a_ref[0], a_ref[1]], axis=1).astype(_f32)
  h = jnp.maximum(s_in_ref[...] + agg * inv, 0.0)
  st = jnp.dot(h, w_ref[...], preferred_element_type=_f32)
  h_out = s_ref.shape[1]
  wh = t_ref.shape[2]
  s_ref[...] = st[:, :h_out] + b_ref[...]
  t_ref[...] = jnp.stack(
      [st[:, h_out:h_out + wh], st[:, h_out + wh:]], axis=0).astype(_bf16)


def _out_body(s_in_ref, a_ref, deg_ref, o_ref):
  inv = 1.0 / jnp.clip(deg_ref[:, 0:1], 1.0, None)
  agg = jnp.concatenate([a_ref[0], a_ref[1]], axis=1).astype(_f32)
  o_ref[...] = s_in_ref[...] + agg * inv


def _tc_first(x, wcat, b2d, h_out):
  k = x.shape[1]
  wh = (wcat.shape[1] - h_out) // 2
  return pl.pallas_call(
      _first_body,
      grid=(N // RB,),
      in_specs=[
          pl.BlockSpec((RB, k), lambda i: (i, 0)),
          pl.BlockSpec(wcat.shape, lambda i: (0, 0)),
          pl.BlockSpec((1, h_out), lambda i: (0, 0)),
      ],
      out_specs=[
          pl.BlockSpec((RB, h_out), lambda i: (i, 0)),
          pl.BlockSpec((NCORES, RB, wh), lambda i: (0, i, 0)),
      ],
      out_shape=[
          jax.ShapeDtypeStruct((N, h_out), _f32),
          jax.ShapeDtypeStruct((NCORES, N, wh), _bf16),
      ],
  )(x, wcat, b2d)


def _tc_mid(s_in, agg, deg8, wcat, b2d, h_out):
  k = s_in.shape[1]
  wa = agg.shape[2]
  wh = (wcat.shape[1] - h_out) // 2
  return pl.pallas_call(
      _mid_body,
      grid=(N // RB,),
      in_specs=[
          pl.BlockSpec((RB, k), lambda i: (i, 0)),
          pl.BlockSpec((NCORES, RB, wa), lambda i: (0, i, 0)),
          pl.BlockSpec((RB, 32), lambda i: (i, 0)),
          pl.BlockSpec(wcat.shape, lambda i: (0, 0)),
          pl.BlockSpec((1, h_out), lambda i: (0, 0)),
      ],
      out_specs=[
          pl.BlockSpec((RB, h_out), lambda i: (i, 0)),
          pl.BlockSpec((NCORES, RB, wh), lambda i: (0, i, 0)),
      ],
      out_shape=[
          jax.ShapeDtypeStruct((N, h_out), _f32),
          jax.ShapeDtypeStruct((NCORES, N, wh), _bf16),
      ],
  )(s_in, agg, deg8, wcat, b2d)


def _tc_out(s_in, agg, deg8):
  k = s_in.shape[1]
  wa = agg.shape[2]
  return pl.pallas_call(
      _out_body,
      grid=(N // RB,),
      in_specs=[
          pl.BlockSpec((RB, k), lambda i: (i, 0)),
          pl.BlockSpec((NCORES, RB, wa), lambda i: (0, i, 0)),
          pl.BlockSpec((RB, 32), lambda i: (i, 0)),
      ],
      out_specs=pl.BlockSpec((RB, k), lambda i: (i, 0)),
      out_shape=jax.ShapeDtypeStruct((N, k), _f32),
  )(s_in, agg, deg8)


@jax.jit
def kernel(x, edge_index, W_self1, W_neigh1, b1, W_self2, W_neigh2, b2,
           W_self3, W_neigh3, b3):
  src = edge_index[0].astype(jnp.int32)
  dst = edge_index[1].astype(jnp.int32)
  srcp = jnp.concatenate([src, jnp.zeros((PADE,), jnp.int32)]).reshape(NS, NCH, CHUNK)
  dstp = jnp.concatenate([dst, jnp.full((PADE,), N, jnp.int32)]).reshape(NS, NCH, CHUNK)
  z128 = jnp.zeros((ZL, 128), _bf16)
  z32 = jnp.zeros((ZL, 32), _bf16)
  z8 = jnp.zeros((ZL, 32), _f32)
  o8 = jnp.ones((CHUNK, 32), _f32)

  wc1 = jnp.concatenate([W_self1, W_neigh1], axis=1)
  wc2 = jnp.concatenate([W_self2, W_neigh2], axis=1)
  wc3 = jnp.concatenate([W_self3, W_neigh3], axis=1)

  s1, t1 = _tc_first(x, wc1, b1[None], 256)
  agg1, deg8 = _make_sc_scatter(128, True)(t1, srcp, dstp, z128, o8, z8)
  s2, t2 = _tc_mid(s1, agg1, deg8, wc2, b2[None], 256)
  (agg2,) = _make_sc_scatter(128)(t2, srcp, dstp, z128)
  s3, t3 = _tc_mid(s2, agg2, deg8, wc3, b3[None], 64)
  (agg3,) = _make_sc_scatter(32)(t3, srcp, dstp, z32)
  return _tc_out(s3, agg3, deg8)


# speedup vs baseline: 1.2866x; 1.0000x over previous
import functools

import jax
import jax.numpy as jnp
from jax import lax
from jax.experimental import pallas as pl
from jax.experimental.pallas import tpu as pltpu
from jax.experimental.pallas import tpu_sc as plsc

N = 10000
E = 160000
NS = 16
NCORES = 2
CHUNK = 120
NCH = 84
EPT = NCH * CHUNK
PADE = NS * EPT - E
NPAD = 10008
RPT = 624
ZL = NPAD - (NS - 1) * RPT
CL = N - (NS - 1) * RPT
RB = 1000

_f32 = jnp.float32
_bf16 = jnp.bfloat16


@functools.cache
def _make_sc_scatter(W, with_deg=False):
  mesh = plsc.VectorSubcoreMesh(core_axis_name="c", subcore_axis_name="s",
                                num_cores=NCORES, num_subcores=NS)
  scratch = [
      pltpu.VMEM((NCH, CHUNK), jnp.int32),
      pltpu.VMEM((NCH, CHUNK), jnp.int32),
      pltpu.VMEM((CHUNK, W), _bf16),
      pltpu.VMEM((CHUNK, W), _bf16),
      pltpu.VMEM_SHARED((NPAD, W), _bf16),
      pltpu.SemaphoreType.DMA,
      pltpu.SemaphoreType.DMA,
      pltpu.SemaphoreType.DMA,
      pltpu.SemaphoreType.DMA,
  ]
  outs = [jax.ShapeDtypeStruct((NCORES, N, W), _bf16)]
  if with_deg:
    outs.append(jax.ShapeDtypeStruct((N, 32), _f32))
    scratch += [
        pltpu.VMEM((CHUNK, 32), _f32),
        pltpu.VMEM_SHARED((NPAD, 32), _f32),
        pltpu.SemaphoreType.DMA,
    ]

  def body(t_hbm, srcp_hbm, dstp_hbm, z_hbm, *rest):
    if with_deg:
      (o_hbm, zd_hbm, out_hbm, deg_hbm, src_v, dst_v, buf0, buf1, acc,
       sem0, sem1, sem2, sem3, ones_v, dacc, sem4) = rest
    else:
      (out_hbm, src_v, dst_v, buf0, buf1, acc,
       sem0, sem1, sem2, sem3) = rest
    core = lax.axis_index("c")
    sub = lax.axis_index("s")

    pltpu.sync_copy(srcp_hbm.at[sub], src_v)
    pltpu.sync_copy(dstp_hbm.at[sub], dst_v)

    @pl.when(sub < NS - 1)
    def _():
      pltpu.sync_copy(z_hbm.at[pl.ds(0, RPT)], acc.at[pl.ds(sub * RPT, RPT)])
    @pl.when(sub == NS - 1)
    def _():
      pltpu.sync_copy(z_hbm, acc.at[pl.ds((NS - 1) * RPT, ZL)])
    if with_deg:
      @pl.when(core == 0)
      def _():
        pltpu.sync_copy(o_hbm, ones_v)

        @pl.when(sub < NS - 1)
        def _():
          pltpu.sync_copy(zd_hbm.at[pl.ds(0, RPT)],
                          dacc.at[pl.ds(sub * RPT, RPT)])
        @pl.when(sub == NS - 1)
        def _():
          pltpu.sync_copy(zd_hbm, dacc.at[pl.ds((NS - 1) * RPT, ZL)])
    plsc.subcore_barrier()

    t_half = t_hbm.at[core]

    def chunk(j2, carry):
      e0 = 2 * j2

      @pl.when(j2 > 0)
      def _():
        pltpu.make_async_copy(buf0, acc.at[dst_v.at[0]], sem2).wait()
        pltpu.make_async_copy(buf1, acc.at[dst_v.at[0]], sem3).wait()
      if with_deg:
        @pl.when((core == 0) & (j2 > 0))
        def _():
          pltpu.make_async_copy(ones_v, dacc.at[dst_v.at[0]], sem4).wait()
          pltpu.make_async_copy(ones_v, dacc.at[dst_v.at[0]], sem4).wait()

      d0 = pltpu.async_copy(t_half.at[src_v.at[e0]], buf0, sem0)
      d1 = pltpu.async_copy(t_half.at[src_v.at[e0 + 1]], buf1, sem1)
      if with_deg:
        @pl.when(core == 0)
        def _():
          pltpu.async_copy(ones_v, dacc.at[dst_v.at[e0]], sem4, add=True)
          pltpu.async_copy(ones_v, dacc.at[dst_v.at[e0 + 1]], sem4, add=True)
      d0.wait()
      pltpu.async_copy(buf0, acc.at[dst_v.at[e0]], sem2, add=True)
      d1.wait()
      pltpu.async_copy(buf1, acc.at[dst_v.at[e0 + 1]], sem3, add=True)
      return carry

    lax.fori_loop(0, NCH // 2, chunk, 0)
    pltpu.make_async_copy(buf0, acc.at[dst_v.at[0]], sem2).wait()
    pltpu.make_async_copy(buf1, acc.at[dst_v.at[0]], sem3).wait()
    if with_deg:
      @pl.when(core == 0)
      def _():
        pltpu.make_async_copy(ones_v, dacc.at[dst_v.at[0]], sem4).wait()
        pltpu.make_async_copy(ones_v, dacc.at[dst_v.at[0]], sem4).wait()
    plsc.subcore_barrier()

    out_half = out_hbm.at[core]

    @pl.when(sub < NS - 1)
    def _():
      pltpu.sync_copy(acc.at[pl.ds(sub * RPT, RPT)],
                      out_half.at[pl.ds(sub * RPT, RPT)])
    @pl.when(sub == NS - 1)
    def _():
      pltpu.sync_copy(acc.at[pl.ds((NS - 1) * RPT, CL)],
                      out_half.at[pl.ds((NS - 1) * RPT, CL)])
    if with_deg:
      @pl.when((core == 0) & (sub < NS - 1))
      def _():
        pltpu.sync_copy(dacc.at[pl.ds(sub * RPT, RPT)],
                        deg_hbm.at[pl.ds(sub * RPT, RPT)])
      @pl.when((core == 0) & (sub == NS - 1))
      def _():
        pltpu.sync_copy(dacc.at[pl.ds((NS - 1) * RPT, CL)],
                        deg_hbm.at[pl.ds((NS - 1) * RPT, CL)])

  return pl.kernel(body, out_type=tuple(outs),
                   mesh=mesh, scratch_types=tuple(scratch),
                   compiler_params=pltpu.CompilerParams(
                       use_tc_tiling_on_sc=False))


def _first_body(x_ref, w_ref, b_ref, s_ref, t_ref):
  st = jnp.dot(x_ref[...], w_ref[...], preferred_element_type=_f32)
  h_out = s_ref.shape[1]
  wh = t_ref.shape[2]
  s_ref[...] = st[:, :h_out] + b_ref[...]
  t_ref[...] = jnp.stack(
      [st[:, h_out:h_out + wh], st[:, h_out + wh:]], axis=0).astype(_bf16)


def _mid_body(s_in_ref, a_ref, deg_ref, w_ref, b_ref, s_ref, t_ref):
  inv = 1.0 / jnp.clip(deg_ref[:, 0:1], 1.0, None)
  agg = jnp.concatenate([a_ref[0], a_ref[1]], axis=1).astype(_f32)
  h = jnp.maximum(s_in_ref[...] + agg * inv, 0.0)
  st = jnp.dot(h, w_ref[...], preferred_element_type=_f32)
  h_out = s_ref.shape[1]
  wh = t_ref.shape[2]
  s_ref[...] = st[:, :h_out] + b_ref[...]
  t_ref[...] = jnp.stack(
      [st[:, h_out:h_out + wh], st[:, h_out + wh:]], axis=0).astype(_bf16)


def _out_body(s_in_ref, a_ref, deg_ref, o_ref):
  inv = 1.0 / jnp.clip(deg_ref[:, 0:1], 1.0, None)
  agg = jnp.concatenate([a_ref[0], a_ref[1]], axis=1).astype(_f32)
  o_ref[...] = s_in_ref[...] + agg * inv


def _tc_first(x, wcat, b2d, h_out):
  k = x.shape[1]
  wh = (wcat.shape[1] - h_out) // 2
  return pl.pallas_call(
      _first_body,
      grid=(N // RB,),
      in_specs=[
          pl.BlockSpec((RB, k), lambda i: (i, 0)),
          pl.BlockSpec(wcat.shape, lambda i: (0, 0)),
          pl.BlockSpec((1, h_out), lambda i: (0, 0)),
      ],
      out_specs=[
          pl.BlockSpec((RB, h_out), lambda i: (i, 0)),
          pl.BlockSpec((NCORES, RB, wh), lambda i: (0, i, 0)),
      ],
      out_shape=[
          jax.ShapeDtypeStruct((N, h_out), _f32),
          jax.ShapeDtypeStruct((NCORES, N, wh), _bf16),
      ],
  )(x, wcat, b2d)


def _tc_mid(s_in, agg, deg8, wcat, b2d, h_out):
  k = s_in.shape[1]
  wa = agg.shape[2]
  wh = (wcat.shape[1] - h_out) // 2
  return pl.pallas_call(
      _mid_body,
      grid=(N // RB,),
      in_specs=[
          pl.BlockSpec((RB, k), lambda i: (i, 0)),
          pl.BlockSpec((NCORES, RB, wa), lambda i: (0, i, 0)),
          pl.BlockSpec((RB, 32), lambda i: (i, 0)),
          pl.BlockSpec(wcat.shape, lambda i: (0, 0)),
          pl.BlockSpec((1, h_out), lambda i: (0, 0)),
      ],
      out_specs=[
          pl.BlockSpec((RB, h_out), lambda i: (i, 0)),
          pl.BlockSpec((NCORES, RB, wh), lambda i: (0, i, 0)),
      ],
      out_shape=[
          jax.ShapeDtypeStruct((N, h_out), _f32),
          jax.ShapeDtypeStruct((NCORES, N, wh), _bf16),
      ],
  )(s_in, agg, deg8, wcat, b2d)


def _tc_out(s_in, agg, deg8):
  k = s_in.shape[1]
  wa = agg.shape[2]
  return pl.pallas_call(
      _out_body,
      grid=(N // RB,),
      in_specs=[
          pl.BlockSpec((RB, k), lambda i: (i, 0)),
          pl.BlockSpec((NCORES, RB, wa), lambda i: (0, i, 0)),
          pl.BlockSpec((RB, 32), lambda i: (i, 0)),
      ],
      out_specs=pl.BlockSpec((RB, k), lambda i: (i, 0)),
      out_shape=jax.ShapeDtypeStruct((N, k), _f32),
  )(s_in, agg, deg8)


@jax.jit
def kernel(x, edge_index, W_self1, W_neigh1, b1, W_self2, W_neigh2, b2,
           W_self3, W_neigh3, b3):
  src = edge_index[0].astype(jnp.int32)
  dst = edge_index[1].astype(jnp.int32)
  srcp = jnp.concatenate([src, jnp.zeros((PADE,), jnp.int32)]).reshape(NS, NCH, CHUNK)
  dstp = jnp.concatenate([dst, jnp.full((PADE,), N, jnp.int32)]).reshape(NS, NCH, CHUNK)
  z128 = jnp.zeros((ZL, 128), _bf16)
  z32 = jnp.zeros((ZL, 32), _bf16)
  z8 = jnp.zeros((ZL, 32), _f32)
  o8 = jnp.ones((CHUNK, 32), _f32)

  wc1 = jnp.concatenate([W_self1, W_neigh1], axis=1)
  wc2 = jnp.concatenate([W_self2, W_neigh2], axis=1)
  wc3 = jnp.concatenate([W_self3, W_neigh3], axis=1)

  s1, t1 = _tc_first(x, wc1, b1[None], 256)
  agg1, deg8 = _make_sc_scatter(128, True)(t1, srcp, dstp, z128, o8, z8)
  s2, t2 = _tc_mid(s1, agg1, deg8, wc2, b2[None], 256)
  (agg2,) = _make_sc_scatter(128)(t2, srcp, dstp, z128)
  s3, t3 = _tc_mid(s2, agg2, deg8, wc3, b3[None], 64)
  (agg3,) = _make_sc_scatter(32)(t3, srcp, dstp, z32)
  return _tc_out(s3, agg3, deg8)
